# baseline (speedup 1.0000x reference)
import functools

import jax
import jax.numpy as jnp
from jax import lax
from jax.experimental import pallas as pl
from jax.experimental.pallas import tpu as pltpu
from jax.experimental.pallas import tpu_sc as plsc

BATCH = 16384
DIM = 64

_info = plsc.get_sparse_core_info()
NC, NS, L = _info.num_cores, _info.num_subcores, _info.num_lanes
NW = NC * NS
BPW = BATCH // NW
GROUPS = BPW // L


def _sc_body(pid_hbm, sid_hbm, ptab_hbm, stab_hbm, w_hbm, b_hbm, out_hbm,
             idxp_v, idxs_v, rows_p, rows_s, w_v, b_v, out_v, sem_p, sem_s):
    wid = lax.axis_index("s") * NC + lax.axis_index("c")
    base = wid * BPW

    pltpu.sync_copy(pid_hbm.at[pl.ds(base, BPW)], idxp_v)
    pltpu.sync_copy(sid_hbm.at[pl.ds(base, BPW)], idxs_v)
    cp = pltpu.async_copy(ptab_hbm.at[idxp_v], rows_p, sem_p)
    cs = pltpu.async_copy(stab_hbm.at[idxs_v], rows_s, sem_s)
    pltpu.sync_copy(w_hbm, w_v)
    pltpu.sync_copy(b_hbm, b_v)
    cp.wait()
    cs.wait()

    bias = b_v[...]
    wp = [w_v[pl.ds(m * L, L)] for m in range(DIM // L)]
    ws = [w_v[pl.ds(DIM + m * L, L)] for m in range(DIM // L)]
    lane = lax.iota(jnp.int32, L)

    def group(g, carry):
        r0 = g * L
        logits = bias
        for j in range(L):
            r = r0 + j
            acc = rows_p[r, pl.ds(0, L)] * wp[0]
            for m in range(1, DIM // L):
                acc = acc + rows_p[r, pl.ds(m * L, L)] * wp[m]
            for m in range(DIM // L):
                acc = acc + rows_s[r, pl.ds(m * L, L)] * ws[m]
            tot = jnp.sum(acc)
            logits = jnp.where(lane == j, logits + tot, logits)
        pred = 1.0 / (1.0 + jnp.exp(-logits))
        out_v[pl.ds(r0, L)] = pred
        return carry

    lax.fori_loop(0, GROUPS, group, 0)
    pltpu.sync_copy(out_v, out_hbm.at[pl.ds(base, BPW)])


@jax.jit
def _run(playlist_ids, song_ids, playlist_table, song_table, w_flat, b_vec):
    mesh = plsc.VectorSubcoreMesh(core_axis_name="c", subcore_axis_name="s")
    call = functools.partial(
        pl.kernel,
        mesh=mesh,
        compiler_params=pltpu.CompilerParams(
            needs_layout_passes=False, use_tc_tiling_on_sc=False),
        out_type=jax.ShapeDtypeStruct((BATCH,), jnp.float32),
        scratch_types=[
            pltpu.VMEM((BPW,), jnp.int32),
            pltpu.VMEM((BPW,), jnp.int32),
            pltpu.VMEM((BPW, DIM), jnp.float32),
            pltpu.VMEM((BPW, DIM), jnp.float32),
            pltpu.VMEM((2 * DIM,), jnp.float32),
            pltpu.VMEM((L,), jnp.float32),
            pltpu.VMEM((BPW,), jnp.float32),
            pltpu.SemaphoreType.DMA,
            pltpu.SemaphoreType.DMA,
        ],
    )(_sc_body)
    return call(playlist_ids, song_ids, playlist_table, song_table,
                w_flat, b_vec)


def kernel(playlist_ids, song_ids, playlist_table, song_table, fc_w, fc_b):
    w_flat = fc_w.reshape(2 * DIM)
    b_vec = jnp.broadcast_to(fc_b.astype(jnp.float32), (L,))
    out = _run(playlist_ids, song_ids, playlist_table, song_table,
               w_flat, b_vec)
    return out.reshape(BATCH, 1)
